# Initial kernel scaffold; baseline (speedup 1.0000x reference)
import jax
import jax.numpy as jnp
from jax.experimental import pallas as pl


def kernel(arg0_1):
    raise NotImplementedError("write your pallas kernel here")


# speedup vs baseline: 10.3799x; 10.3799x over previous
import functools

import jax
import jax.numpy as jnp
from jax import lax
from jax.experimental import pallas as pl
from jax.experimental.pallas import tpu as pltpu
from jax.experimental.pallas import tpu_sc as plsc

_NUM_EDGES = 13264
_NUM_NODES = 2708

_NC = 2
_NS = 16
_L = 16

_E_PAD = 13312
_E3 = _E_PAD // (_NC * _NS)
_E1 = _E_PAD // _NS
_W1 = 64
_NW1 = _E1 // _W1
_N_PAD = 2816
_NZ = _N_PAD // _NS
_SENT = _NUM_NODES


def _rsqrt(x):
    i = lax.bitcast_convert_type(x, jnp.int32)
    i = jnp.int32(0x5F3759DF) - lax.shift_right_logical(i, 1)
    y = lax.bitcast_convert_type(i, jnp.float32)
    for _ in range(3):
        y = y * (jnp.float32(1.5) - jnp.float32(0.5) * x * y * y)
    return y


@functools.partial(
    pl.kernel,
    out_type=jax.ShapeDtypeStruct((_NC * _NS, _E3), jnp.float32),
    mesh=plsc.VectorSubcoreMesh(
        core_axis_name="c", subcore_axis_name="s",
        num_cores=_NC, num_subcores=_NS),
    scratch_types=[
        pltpu.VMEM_SHARED((_N_PAD,), jnp.float32),
        pltpu.VMEM((_NW1, _W1), jnp.int32),
        pltpu.VMEM((_W1,), jnp.float32),
        pltpu.VMEM((_NZ,), jnp.float32),
        pltpu.VMEM((_N_PAD,), jnp.float32),
        pltpu.VMEM((_E3,), jnp.int32),
        pltpu.VMEM((_E3,), jnp.int32),
        pltpu.VMEM((_E3,), jnp.float32),
    ],
    compiler_params=pltpu.CompilerParams(needs_layout_passes=False),
)
def _norm_kernel(d1_hbm, s3_hbm, d3_hbm, out_hbm,
                 deg_sh, idx1_v, ones_v, zeros_v, deg_v, s3_v, d3_v, out_v):
    c = lax.axis_index("c")
    s = lax.axis_index("s")
    wid = c * _NS + s

    ones16 = jnp.full((_L,), 1.0, jnp.float32)
    zeros16 = jnp.zeros((_L,), jnp.float32)
    for k in range(_W1 // _L):
        ones_v[pl.ds(k * _L, _L)] = ones16
    for k in range(_NZ // _L):
        zeros_v[pl.ds(k * _L, _L)] = zeros16

    pltpu.sync_copy(zeros_v, deg_sh.at[pl.ds(s * _NZ, _NZ)])
    plsc.subcore_barrier()

    pltpu.sync_copy(d1_hbm.at[s], idx1_v)
    for j in range(_NW1):
        pltpu.sync_copy(ones_v, deg_sh.at[idx1_v.at[j]], add=True)
    plsc.subcore_barrier()

    pltpu.sync_copy(deg_sh, deg_v)

    pltpu.sync_copy(s3_hbm.at[wid], s3_v)
    pltpu.sync_copy(d3_hbm.at[wid], d3_v)
    for i in range(_E3 // _L):
        si = s3_v[pl.ds(i * _L, _L)]
        di = d3_v[pl.ds(i * _L, _L)]
        a = plsc.load_gather(deg_v, [si])
        b = plsc.load_gather(deg_v, [di])
        m = a * b
        y = _rsqrt(m)
        out_v[pl.ds(i * _L, _L)] = jnp.where(m > 0.0, y, jnp.float32(0.0))
    pltpu.sync_copy(out_v, out_hbm.at[wid])


def kernel(arg0_1):
    e = arg0_1.astype(jnp.int32)
    pad = jnp.full((_E_PAD - _NUM_EDGES,), _SENT, jnp.int32)
    srcp = jnp.concatenate([e[0], pad])
    dstp = jnp.concatenate([e[1], pad])
    d1 = dstp.reshape(_NS, _NW1, _W1)
    s3 = srcp.reshape(_NC * _NS, _E3)
    d3 = dstp.reshape(_NC * _NS, _E3)
    out = _norm_kernel(d1, s3, d3)
    return (out.reshape(-1)[:_NUM_EDGES],)


# speedup vs baseline: 11.1593x; 1.0751x over previous
import functools

import jax
import jax.numpy as jnp
from jax import lax
from jax.experimental import pallas as pl
from jax.experimental.pallas import tpu as pltpu
from jax.experimental.pallas import tpu_sc as plsc

_NUM_EDGES = 13264
_NUM_NODES = 2708

_NC = 2
_NS = 16
_L = 16

_E1 = 832
_W1 = 64
_NW1 = _E1 // _W1
_E1T = _NUM_EDGES - 15 * _E1

_E3 = 416
_OFF_LAST = _NUM_EDGES - _E3

_N_PAD = 2816
_NZ = _N_PAD // _NS


def _rsqrt(x):
    i = lax.bitcast_convert_type(x, jnp.int32)
    i = jnp.int32(0x5F3759DF) - lax.shift_right_logical(i, 1)
    y = lax.bitcast_convert_type(i, jnp.float32)
    for _ in range(3):
        y = y * (jnp.float32(1.5) - jnp.float32(0.5) * x * y * y)
    return y


@functools.partial(
    pl.kernel,
    out_type=jax.ShapeDtypeStruct((_NUM_EDGES,), jnp.float32),
    mesh=plsc.VectorSubcoreMesh(
        core_axis_name="c", subcore_axis_name="s",
        num_cores=_NC, num_subcores=_NS),
    scratch_types=[
        pltpu.VMEM_SHARED((_N_PAD,), jnp.float32),
        pltpu.VMEM((_E1,), jnp.int32),
        pltpu.VMEM((_W1,), jnp.float32),
        pltpu.VMEM((_NZ,), jnp.float32),
        pltpu.VMEM((_N_PAD,), jnp.float32),
        pltpu.VMEM((_E3,), jnp.int32),
        pltpu.VMEM((_E3,), jnp.int32),
        pltpu.VMEM((_E3,), jnp.float32),
    ],
    compiler_params=pltpu.CompilerParams(
        needs_layout_passes=False, use_tc_tiling_on_sc=False),
)
def _norm_kernel(e_hbm, out_hbm,
                 deg_sh, idx1_v, ones_v, zeros_v, deg_v, s3_v, d3_v, out_v):
    c = lax.axis_index("c")
    s = lax.axis_index("s")
    wid = c * _NS + s

    ones16 = jnp.full((_L,), 1.0, jnp.float32)
    zeros16 = jnp.zeros((_L,), jnp.float32)
    for k in range(_W1 // _L):
        ones_v[pl.ds(k * _L, _L)] = ones16
    for k in range(_NZ // _L):
        zeros_v[pl.ds(k * _L, _L)] = zeros16

    pltpu.sync_copy(zeros_v, deg_sh.at[pl.ds(s * _NZ, _NZ)])

    @pl.when(s < _NS - 1)
    def _():
        pltpu.sync_copy(e_hbm.at[1, pl.ds(s * _E1, _E1)], idx1_v)

    @pl.when(s == _NS - 1)
    def _():
        pltpu.sync_copy(e_hbm.at[1, pl.ds(15 * _E1, _E1T)],
                        idx1_v.at[pl.ds(0, _E1T)])

    plsc.subcore_barrier()

    for j in range(_NW1 - 1):
        pltpu.sync_copy(ones_v, deg_sh.at[idx1_v.at[pl.ds(j * _W1, _W1)]],
                        add=True)

    @pl.when(s < _NS - 1)
    def _():
        pltpu.sync_copy(ones_v, deg_sh.at[idx1_v.at[pl.ds(12 * _W1, _W1)]],
                        add=True)

    @pl.when(s == _NS - 1)
    def _():
        pltpu.sync_copy(ones_v.at[pl.ds(0, _L)],
                        deg_sh.at[idx1_v.at[pl.ds(12 * _W1, _L)]], add=True)

    plsc.subcore_barrier()

    pltpu.sync_copy(deg_sh, deg_v)

    off = jnp.where(wid == _NC * _NS - 1, _OFF_LAST, wid * _E3)
    pltpu.sync_copy(e_hbm.at[0, pl.ds(off, _E3)], s3_v)
    pltpu.sync_copy(e_hbm.at[1, pl.ds(off, _E3)], d3_v)
    for i in range(_E3 // _L):
        si = s3_v[pl.ds(i * _L, _L)]
        di = d3_v[pl.ds(i * _L, _L)]
        a = plsc.load_gather(deg_v, [si])
        b = plsc.load_gather(deg_v, [di])
        m = a * b
        y = _rsqrt(m)
        out_v[pl.ds(i * _L, _L)] = jnp.where(m > 0.0, y, jnp.float32(0.0))
    pltpu.sync_copy(out_v, out_hbm.at[pl.ds(off, _E3)])


def kernel(arg0_1):
    return (_norm_kernel(arg0_1.astype(jnp.int32)),)


# speedup vs baseline: 11.6288x; 1.0421x over previous
import functools

import jax
import jax.numpy as jnp
from jax import lax
from jax.experimental import pallas as pl
from jax.experimental.pallas import tpu as pltpu
from jax.experimental.pallas import tpu_sc as plsc

_NUM_EDGES = 13264
_NUM_NODES = 2708

_NC = 2
_NS = 16
_L = 16

_E1 = 832
_W1 = 64
_NW1 = _E1 // _W1
_E1T = _NUM_EDGES - 15 * _E1

_E3 = 416
_OFF_LAST = _NUM_EDGES - _E3

_N_PAD = 2816
_NZ = _N_PAD // _NS


def _rsqrt(x):
    i = lax.bitcast_convert_type(x, jnp.int32)
    i = jnp.int32(0x5F3759DF) - lax.shift_right_logical(i, 1)
    y = lax.bitcast_convert_type(i, jnp.float32)
    for _ in range(3):
        y = y * (jnp.float32(1.5) - jnp.float32(0.5) * x * y * y)
    return y


@functools.partial(
    pl.kernel,
    out_type=jax.ShapeDtypeStruct((_NUM_EDGES,), jnp.float32),
    mesh=plsc.VectorSubcoreMesh(
        core_axis_name="c", subcore_axis_name="s",
        num_cores=_NC, num_subcores=_NS),
    scratch_types=[
        pltpu.VMEM_SHARED((_N_PAD,), jnp.float32),
        pltpu.VMEM((_E1,), jnp.int32),
        pltpu.VMEM((_W1,), jnp.float32),
        pltpu.VMEM((_NZ,), jnp.float32),
        pltpu.VMEM((_N_PAD,), jnp.float32),
        pltpu.VMEM((_E3,), jnp.int32),
        pltpu.VMEM((_E3,), jnp.int32),
        pltpu.VMEM((_E3,), jnp.float32),
    ],
    compiler_params=pltpu.CompilerParams(
        needs_layout_passes=False, use_tc_tiling_on_sc=False),
)
def _norm_kernel(e_hbm, out_hbm,
                 deg_sh, idx1_v, ones_v, zeros_v, deg_v, s3_v, d3_v, out_v):
    c = lax.axis_index("c")
    s = lax.axis_index("s")
    wid = c * _NS + s

    ones16 = jnp.full((_L,), 1.0, jnp.float32)
    zeros16 = jnp.zeros((_L,), jnp.float32)
    for k in range(_W1 // _L):
        ones_v[pl.ds(k * _L, _L)] = ones16
    for k in range(_NZ // _L):
        zeros_v[pl.ds(k * _L, _L)] = zeros16

    pltpu.sync_copy(zeros_v, deg_sh.at[pl.ds(s * _NZ, _NZ)])

    @pl.when(s < _NS - 1)
    def _():
        pltpu.sync_copy(e_hbm.at[1, pl.ds(s * _E1, _E1)], idx1_v)

    @pl.when(s == _NS - 1)
    def _():
        pltpu.sync_copy(e_hbm.at[1, pl.ds(15 * _E1, _E1T)],
                        idx1_v.at[pl.ds(0, _E1T)])

    plsc.subcore_barrier()

    def _scatter_window(j, carry):
        pltpu.sync_copy(ones_v, deg_sh.at[idx1_v.at[pl.ds(j * _W1, _W1)]],
                        add=True)
        return carry

    lax.fori_loop(0, _NW1 - 1, _scatter_window, 0)

    @pl.when(s < _NS - 1)
    def _():
        pltpu.sync_copy(ones_v, deg_sh.at[idx1_v.at[pl.ds(12 * _W1, _W1)]],
                        add=True)

    @pl.when(s == _NS - 1)
    def _():
        pltpu.sync_copy(ones_v.at[pl.ds(0, _L)],
                        deg_sh.at[idx1_v.at[pl.ds(12 * _W1, _L)]], add=True)

    plsc.subcore_barrier()

    pltpu.sync_copy(deg_sh, deg_v)

    off = jnp.where(wid == _NC * _NS - 1, _OFF_LAST, wid * _E3)
    pltpu.sync_copy(e_hbm.at[0, pl.ds(off, _E3)], s3_v)
    pltpu.sync_copy(e_hbm.at[1, pl.ds(off, _E3)], d3_v)
    def _edge_step(i, carry):
        si = s3_v[pl.ds(i * _L, _L)]
        di = d3_v[pl.ds(i * _L, _L)]
        a = plsc.load_gather(deg_v, [si])
        b = plsc.load_gather(deg_v, [di])
        m = a * b
        y = _rsqrt(m)
        out_v[pl.ds(i * _L, _L)] = jnp.where(m > 0.0, y, jnp.float32(0.0))
        return carry

    lax.fori_loop(0, _E3 // _L, _edge_step, 0)
    pltpu.sync_copy(out_v, out_hbm.at[pl.ds(off, _E3)])


def kernel(arg0_1):
    return (_norm_kernel(arg0_1.astype(jnp.int32)),)


# speedup vs baseline: 12.6645x; 1.0891x over previous
import functools

import jax
import jax.numpy as jnp
from jax import lax
from jax.experimental import pallas as pl
from jax.experimental.pallas import tpu as pltpu
from jax.experimental.pallas import tpu_sc as plsc

_NUM_EDGES = 13264
_NUM_NODES = 2708

_NC = 2
_NS = 16
_L = 16

_E1 = 832
_W1 = 128
_NW1 = _E1 // _W1
_WLAST = _E1 - _NW1 * _W1
_E1T = _NUM_EDGES - 15 * _E1
_WLASTT = _E1T - _NW1 * _W1

_E3 = 416
_OFF_LAST = _NUM_EDGES - _E3

_N_PAD = 2816
_NZ = _N_PAD // _NS


def _rsqrt(x):
    i = lax.bitcast_convert_type(x, jnp.int32)
    i = jnp.int32(0x5F3759DF) - lax.shift_right_logical(i, 1)
    y = lax.bitcast_convert_type(i, jnp.float32)
    for _ in range(3):
        y = y * (jnp.float32(1.5) - jnp.float32(0.5) * x * y * y)
    return y


@functools.partial(
    pl.kernel,
    out_type=jax.ShapeDtypeStruct((_NUM_EDGES,), jnp.float32),
    mesh=plsc.VectorSubcoreMesh(
        core_axis_name="c", subcore_axis_name="s",
        num_cores=_NC, num_subcores=_NS),
    scratch_types=[
        pltpu.VMEM_SHARED((_N_PAD,), jnp.float32),
        pltpu.VMEM((_E1,), jnp.int32),
        pltpu.VMEM((_W1,), jnp.float32),
        pltpu.VMEM((_NZ,), jnp.float32),
        pltpu.VMEM((_N_PAD,), jnp.float32),
        pltpu.VMEM((_E3,), jnp.int32),
        pltpu.VMEM((_E3,), jnp.int32),
        pltpu.VMEM((_E3,), jnp.float32),
        pltpu.SemaphoreType.DMA,
        pltpu.SemaphoreType.DMA,
    ],
    compiler_params=pltpu.CompilerParams(
        needs_layout_passes=False, use_tc_tiling_on_sc=False),
)
def _norm_kernel(e_hbm, out_hbm,
                 deg_sh, idx1_v, ones_v, zeros_v, deg_v, s3_v, d3_v, out_v,
                 sem_in, sem_sc):
    c = lax.axis_index("c")
    s = lax.axis_index("s")
    wid = c * _NS + s

    off3 = jnp.where(wid == _NC * _NS - 1, _OFF_LAST, wid * _E3)
    ld_s3 = pltpu.async_copy(e_hbm.at[0, pl.ds(off3, _E3)], s3_v, sem_in)
    ld_d3 = pltpu.async_copy(e_hbm.at[1, pl.ds(off3, _E3)], d3_v, sem_in)
    off1 = jnp.where(s == _NS - 1, _NUM_EDGES - _E1, s * _E1)
    base1 = jnp.where(s == _NS - 1, _E1 - _E1T, 0)
    ld_i1 = pltpu.async_copy(e_hbm.at[1, pl.ds(off1, _E1)], idx1_v, sem_in)

    ones16 = jnp.full((_L,), 1.0, jnp.float32)
    zeros16 = jnp.zeros((_L,), jnp.float32)
    for k in range(_W1 // _L):
        ones_v[pl.ds(k * _L, _L)] = ones16
    for k in range(_NZ // _L):
        zeros_v[pl.ds(k * _L, _L)] = zeros16

    pltpu.sync_copy(zeros_v, deg_sh.at[pl.ds(s * _NZ, _NZ)])
    ld_i1.wait()
    plsc.subcore_barrier()

    scs = []
    for j in range(_NW1):
        scs.append(pltpu.async_copy(
            ones_v, deg_sh.at[idx1_v.at[pl.ds(base1 + j * _W1, _W1)]], sem_sc,
            add=True))

    @pl.when(s < _NS - 1)
    def _():
        d = pltpu.async_copy(
            ones_v.at[pl.ds(0, _WLAST)],
            deg_sh.at[idx1_v.at[pl.ds(_NW1 * _W1, _WLAST)]], sem_sc, add=True)
        d.wait()

    @pl.when(s == _NS - 1)
    def _():
        d = pltpu.async_copy(
            ones_v.at[pl.ds(0, _WLASTT)],
            deg_sh.at[idx1_v.at[pl.ds(_E1 - _WLASTT, _WLASTT)]], sem_sc,
            add=True)
        d.wait()

    for d in scs:
        d.wait()
    plsc.subcore_barrier()

    pltpu.sync_copy(deg_sh, deg_v)
    ld_s3.wait()
    ld_d3.wait()

    def _edge_step(i, carry):
        si = s3_v[pl.ds(i * _L, _L)]
        di = d3_v[pl.ds(i * _L, _L)]
        a = plsc.load_gather(deg_v, [si])
        b = plsc.load_gather(deg_v, [di])
        m = a * b
        y = _rsqrt(m)
        out_v[pl.ds(i * _L, _L)] = jnp.where(m > 0.0, y, jnp.float32(0.0))
        return carry

    lax.fori_loop(0, _E3 // _L, _edge_step, 0)
    pltpu.sync_copy(out_v, out_hbm.at[pl.ds(off3, _E3)])


def kernel(arg0_1):
    return (_norm_kernel(arg0_1.astype(jnp.int32)),)


# speedup vs baseline: 12.8555x; 1.0151x over previous
import functools

import jax
import jax.numpy as jnp
from jax import lax
from jax.experimental import pallas as pl
from jax.experimental.pallas import tpu as pltpu
from jax.experimental.pallas import tpu_sc as plsc

_NUM_EDGES = 13264
_NUM_NODES = 2708

_NC = 2
_NS = 16
_L = 16

_E1 = 832
_E1T = _NUM_EDGES - 15 * _E1

_E3 = 416
_OFF_LAST = _NUM_EDGES - _E3

_N_PAD = 2816
_NZ = _N_PAD // _NS


def _rsqrt(x):
    i = lax.bitcast_convert_type(x, jnp.int32)
    i = jnp.int32(0x5F3759DF) - lax.shift_right_logical(i, 1)
    y = lax.bitcast_convert_type(i, jnp.float32)
    for _ in range(2):
        y = y * (jnp.float32(1.5) - jnp.float32(0.5) * x * y * y)
    return y


@functools.partial(
    pl.kernel,
    out_type=jax.ShapeDtypeStruct((_NUM_EDGES,), jnp.float32),
    mesh=plsc.VectorSubcoreMesh(
        core_axis_name="c", subcore_axis_name="s",
        num_cores=_NC, num_subcores=_NS),
    scratch_types=[
        pltpu.VMEM_SHARED((_N_PAD,), jnp.float32),
        pltpu.VMEM((_E1,), jnp.int32),
        pltpu.VMEM((_E1,), jnp.float32),
        pltpu.VMEM((_NZ,), jnp.float32),
        pltpu.VMEM((_N_PAD,), jnp.float32),
        pltpu.VMEM((_E3,), jnp.int32),
        pltpu.VMEM((_E3,), jnp.int32),
        pltpu.VMEM((_E3,), jnp.float32),
        pltpu.SemaphoreType.DMA,
    ],
    compiler_params=pltpu.CompilerParams(
        needs_layout_passes=False, use_tc_tiling_on_sc=False),
)
def _norm_kernel(e_hbm, out_hbm,
                 deg_sh, idx1_v, ones_v, zeros_v, deg_v, s3_v, d3_v, out_v,
                 sem_in):
    c = lax.axis_index("c")
    s = lax.axis_index("s")
    wid = c * _NS + s

    off3 = jnp.where(wid == _NC * _NS - 1, _OFF_LAST, wid * _E3)
    ld_s3 = pltpu.async_copy(e_hbm.at[0, pl.ds(off3, _E3)], s3_v, sem_in)
    ld_d3 = pltpu.async_copy(e_hbm.at[1, pl.ds(off3, _E3)], d3_v, sem_in)
    off1 = jnp.where(s == _NS - 1, _NUM_EDGES - _E1, s * _E1)
    ld_i1 = pltpu.async_copy(e_hbm.at[1, pl.ds(off1, _E1)], idx1_v, sem_in)

    ones16 = jnp.full((_L,), 1.0, jnp.float32)
    zeros16 = jnp.zeros((_L,), jnp.float32)

    def _fill_ones(k, carry):
        ones_v[pl.ds(k * _L, _L)] = ones16
        return carry

    lax.fori_loop(0, _E1 // _L, _fill_ones, 0)
    for k in range(_NZ // _L):
        zeros_v[pl.ds(k * _L, _L)] = zeros16

    pltpu.sync_copy(zeros_v, deg_sh.at[pl.ds(s * _NZ, _NZ)])
    ld_i1.wait()
    plsc.subcore_barrier()

    @pl.when(s < _NS - 1)
    def _():
        pltpu.sync_copy(ones_v, deg_sh.at[idx1_v], add=True)

    @pl.when(s == _NS - 1)
    def _():
        pltpu.sync_copy(ones_v.at[pl.ds(0, _E1T)],
                        deg_sh.at[idx1_v.at[pl.ds(_E1 - _E1T, _E1T)]],
                        add=True)

    plsc.subcore_barrier()

    pltpu.sync_copy(deg_sh, deg_v)
    ld_s3.wait()
    ld_d3.wait()

    def _edge_step(i, carry):
        si = s3_v[pl.ds(i * _L, _L)]
        di = d3_v[pl.ds(i * _L, _L)]
        a = plsc.load_gather(deg_v, [si])
        b = plsc.load_gather(deg_v, [di])
        m = a * b
        y = _rsqrt(m)
        out_v[pl.ds(i * _L, _L)] = jnp.where(m > 0.0, y, jnp.float32(0.0))
        return carry

    lax.fori_loop(0, _E3 // _L, _edge_step, 0)
    pltpu.sync_copy(out_v, out_hbm.at[pl.ds(off3, _E3)])


def kernel(arg0_1):
    return (_norm_kernel(arg0_1.astype(jnp.int32)),)


# speedup vs baseline: 12.9754x; 1.0093x over previous
import functools

import jax
import jax.numpy as jnp
from jax import lax
from jax.experimental import pallas as pl
from jax.experimental.pallas import tpu as pltpu
from jax.experimental.pallas import tpu_sc as plsc

_NUM_EDGES = 13264
_NUM_NODES = 2708

_NC = 2
_NS = 16
_L = 16

_E1 = 832
_E1T = _NUM_EDGES - 15 * _E1

_E3 = 416
_OFF_LAST = _NUM_EDGES - _E3

_N_PAD = 2816
_NZ = _N_PAD // _NS


def _rsqrt(x):
    i = lax.bitcast_convert_type(x, jnp.int32)
    i = jnp.int32(0x5F3759DF) - lax.shift_right_logical(i, 1)
    y = lax.bitcast_convert_type(i, jnp.float32)
    for _ in range(2):
        y = y * (jnp.float32(1.5) - jnp.float32(0.5) * x * y * y)
    return y


@functools.partial(
    pl.kernel,
    out_type=jax.ShapeDtypeStruct((_NUM_EDGES,), jnp.float32),
    mesh=plsc.VectorSubcoreMesh(
        core_axis_name="c", subcore_axis_name="s",
        num_cores=_NC, num_subcores=_NS),
    scratch_types=[
        pltpu.VMEM_SHARED((_N_PAD,), jnp.float32),
        pltpu.VMEM((_E1,), jnp.int32),
        pltpu.VMEM((_E1,), jnp.float32),
        pltpu.VMEM((_NZ,), jnp.float32),
        pltpu.VMEM((_N_PAD,), jnp.float32),
        pltpu.VMEM((_E3,), jnp.int32),
        pltpu.VMEM((_E3,), jnp.int32),
        pltpu.VMEM((_E3,), jnp.float32),
        pltpu.SemaphoreType.DMA,
    ],
    compiler_params=pltpu.CompilerParams(
        needs_layout_passes=False, use_tc_tiling_on_sc=False),
)
def _norm_kernel(e_hbm, out_hbm,
                 deg_sh, idx1_v, ones_v, zeros_v, deg_v, s3_v, d3_v, out_v,
                 sem_in):
    c = lax.axis_index("c")
    s = lax.axis_index("s")
    wid = c * _NS + s

    off3 = jnp.where(wid == _NC * _NS - 1, _OFF_LAST, wid * _E3)
    ld_s3 = pltpu.async_copy(e_hbm.at[0, pl.ds(off3, _E3)], s3_v, sem_in)
    ld_d3 = pltpu.async_copy(e_hbm.at[1, pl.ds(off3, _E3)], d3_v, sem_in)
    off1 = jnp.where(s == _NS - 1, _NUM_EDGES - _E1, s * _E1)
    ld_i1 = pltpu.async_copy(e_hbm.at[1, pl.ds(off1, _E1)], idx1_v, sem_in)

    ones16 = jnp.full((_L,), 1.0, jnp.float32)
    zeros16 = jnp.zeros((_L,), jnp.float32)

    def _fill_ones(k, carry):
        ones_v[pl.ds(k * _L, _L)] = ones16
        return carry

    lax.fori_loop(0, _E1 // _L, _fill_ones, 0)
    for k in range(_NZ // _L):
        zeros_v[pl.ds(k * _L, _L)] = zeros16

    pltpu.sync_copy(zeros_v, deg_sh.at[pl.ds(s * _NZ, _NZ)])
    ld_i1.wait()
    plsc.subcore_barrier()

    @pl.when(s < _NS - 1)
    def _():
        pltpu.sync_copy(ones_v, deg_sh.at[idx1_v], add=True)

    @pl.when(s == _NS - 1)
    def _():
        pltpu.sync_copy(ones_v.at[pl.ds(0, _E1T)],
                        deg_sh.at[idx1_v.at[pl.ds(_E1 - _E1T, _E1T)]],
                        add=True)

    plsc.subcore_barrier()

    pltpu.sync_copy(deg_sh, deg_v)
    ld_s3.wait()
    ld_d3.wait()

    @plsc.parallel_loop(0, _E3, step=_L, unroll=2)
    def _edge_step(i):
        si = s3_v[pl.ds(i, _L)]
        di = d3_v[pl.ds(i, _L)]
        a = plsc.load_gather(deg_v, [si])
        b = plsc.load_gather(deg_v, [di])
        m = a * b
        y = _rsqrt(m)
        out_v[pl.ds(i, _L)] = jnp.where(m > 0.0, y, jnp.float32(0.0))
    pltpu.sync_copy(out_v, out_hbm.at[pl.ds(off3, _E3)])


def kernel(arg0_1):
    return (_norm_kernel(arg0_1.astype(jnp.int32)),)
